# Initial kernel scaffold; baseline (speedup 1.0000x reference)
import jax
import jax.numpy as jnp
from jax.experimental import pallas as pl


def kernel(x, op_norm_w, ffn_norm_w, w_in, w_conv, w_out, w_router, expert_bias, w_g, w_u, w_d):
    raise NotImplementedError("write your pallas kernel here")


# speedup vs baseline: 2.3102x; 2.3102x over previous
import functools

import jax
import jax.numpy as jnp
from jax.experimental import pallas as pl
from jax.experimental.pallas import tpu as pltpu

_B, _L, _H = 1, 2048, 768
_E, _TOP_K, _FF = 16, 2, 512
_KCONV = 3
_EPS = 1e-05

_BLK1 = 256


def _rms(x, w):
    return x * jax.lax.rsqrt(jnp.mean(x * x, axis=-1, keepdims=True) + _EPS) * w


def _stage1_body(x_ref, opw_ref, ffnw_ref, win_ref, wconvt_ref, wout_ref,
                 wr_ref, bias_ref, h_ref, t_ref, comb_ref, carry_ref):
    i = pl.program_id(0)

    @pl.when(i == 0)
    def _():
        carry_ref[...] = jnp.zeros_like(carry_ref)

    xb = x_ref[...]
    hn = _rms(xb, opw_ref[...])
    bcx = jnp.dot(hn.astype(jnp.bfloat16), win_ref[...],
                  preferred_element_type=jnp.float32)
    b_ = bcx[:, :_H]
    c_ = bcx[:, _H:2 * _H]
    xc = bcx[:, 2 * _H:]
    Bx = b_ * xc

    carry = carry_ref[...]
    sh1 = jnp.concatenate([carry[7:8], Bx[:-1]], axis=0)
    sh2 = jnp.concatenate([carry[6:8], Bx[:-2]], axis=0)[:_BLK1]
    w0 = wconvt_ref[0:1, :]
    w1 = wconvt_ref[1:2, :]
    w2 = wconvt_ref[2:3, :]
    conv = sh2 * w0 + sh1 * w1 + Bx * w2
    carry_ref[...] = Bx[-8:]

    y = c_ * conv
    r = jnp.dot(y.astype(jnp.bfloat16), wout_ref[...],
                preferred_element_type=jnp.float32)
    h = xb + r
    h_ref[...] = h

    t = _rms(h, ffnw_ref[...])
    t_ref[...] = t.astype(jnp.bfloat16)

    logits = jnp.dot(t, wr_ref[...], preferred_element_type=jnp.float32)
    m = jnp.max(logits, axis=-1, keepdims=True)
    p = jnp.exp(logits - m)
    g = p / jnp.sum(p, axis=-1, keepdims=True) + bias_ref[...]

    iota = jax.lax.broadcasted_iota(jnp.int32, g.shape, 1)
    m0 = jnp.max(g, axis=-1, keepdims=True)
    i0 = jnp.min(jnp.where(g == m0, iota, _E), axis=-1, keepdims=True)
    oh0 = iota == i0
    g1 = jnp.where(oh0, -jnp.inf, g)
    m1 = jnp.max(g1, axis=-1, keepdims=True)
    i1 = jnp.min(jnp.where(g1 == m1, iota, _E), axis=-1, keepdims=True)
    oh1 = iota == i1
    denom = m0 + m1 + 1e-20
    comb_ref[...] = (jnp.where(oh0, m0, 0.0) + jnp.where(oh1, m1, 0.0)) / denom


def _stage2_body(t_ref, comb_ref, h_ref, wg_ref, wu_ref, wd_ref, out_ref):
    e = pl.program_id(0)
    t = t_ref[...]
    a = jnp.dot(t, wg_ref[0], preferred_element_type=jnp.float32)
    u = jnp.dot(t, wu_ref[0], preferred_element_type=jnp.float32)
    he = (a / (1.0 + jnp.exp(-a))) * u
    contrib = jnp.dot(he.astype(jnp.bfloat16), wd_ref[0],
                      preferred_element_type=jnp.float32)
    comb = comb_ref[...]
    lane = jax.lax.broadcasted_iota(jnp.int32, comb.shape, 1)
    ccol = jnp.sum(jnp.where(lane == e, comb, 0.0), axis=-1, keepdims=True)

    @pl.when(e == 0)
    def _():
        out_ref[...] = h_ref[...]

    out_ref[...] += ccol * contrib


def kernel(x, op_norm_w, ffn_norm_w, w_in, w_conv, w_out, w_router,
           expert_bias, w_g, w_u, w_d):
    x2 = x.reshape(_L, _H)
    wconvt = jnp.zeros((8, _H), jnp.float32).at[:_KCONV].set(w_conv.T)

    nblk = _L // _BLK1
    h, t, comb = pl.pallas_call(
        _stage1_body,
        grid=(nblk,),
        in_specs=[
            pl.BlockSpec((_BLK1, _H), lambda i: (i, 0)),
            pl.BlockSpec((1, _H), lambda i: (0, 0)),
            pl.BlockSpec((1, _H), lambda i: (0, 0)),
            pl.BlockSpec((_H, 3 * _H), lambda i: (0, 0)),
            pl.BlockSpec((8, _H), lambda i: (0, 0)),
            pl.BlockSpec((_H, _H), lambda i: (0, 0)),
            pl.BlockSpec((_H, _E), lambda i: (0, 0)),
            pl.BlockSpec((1, _E), lambda i: (0, 0)),
        ],
        out_specs=[
            pl.BlockSpec((_BLK1, _H), lambda i: (i, 0)),
            pl.BlockSpec((_BLK1, _H), lambda i: (i, 0)),
            pl.BlockSpec((_BLK1, _E), lambda i: (i, 0)),
        ],
        out_shape=[
            jax.ShapeDtypeStruct((_L, _H), jnp.float32),
            jax.ShapeDtypeStruct((_L, _H), jnp.bfloat16),
            jax.ShapeDtypeStruct((_L, _E), jnp.float32),
        ],
        scratch_shapes=[pltpu.VMEM((8, _H), jnp.float32)],
    )(
        x2,
        op_norm_w.reshape(1, _H),
        ffn_norm_w.reshape(1, _H),
        w_in.astype(jnp.bfloat16),
        wconvt,
        w_out.astype(jnp.bfloat16),
        w_router,
        expert_bias.reshape(1, _E),
    )

    out = pl.pallas_call(
        _stage2_body,
        grid=(_E,),
        in_specs=[
            pl.BlockSpec((_L, _H), lambda e: (0, 0)),
            pl.BlockSpec((_L, _E), lambda e: (0, 0)),
            pl.BlockSpec((_L, _H), lambda e: (0, 0)),
            pl.BlockSpec((1, _H, _FF), lambda e: (e, 0, 0)),
            pl.BlockSpec((1, _H, _FF), lambda e: (e, 0, 0)),
            pl.BlockSpec((1, _FF, _H), lambda e: (e, 0, 0)),
        ],
        out_specs=pl.BlockSpec((_L, _H), lambda e: (0, 0)),
        out_shape=jax.ShapeDtypeStruct((_L, _H), jnp.float32),
    )(t, comb, h,
      w_g.astype(jnp.bfloat16), w_u.astype(jnp.bfloat16),
      w_d.astype(jnp.bfloat16))

    return out.reshape(_B, _L, _H)
